# baseline (speedup 1.0000x reference)
import functools

import jax
import jax.numpy as jnp
from jax import lax
from jax.experimental import pallas as pl
from jax.experimental.pallas import tpu as pltpu
from jax.experimental.pallas import tpu_sc as plsc

_NC = 2
_NS = 16
_NW = _NC * _NS
_L = 16
_R = 16
_NBUF = 8
_LEAD = 4


def kernel(embeddings, pos_table):
    B, P, H = embeddings.shape
    ppw = P // _NW
    nchunk = ppw // _R
    nvec = (_R * H) // _L

    mesh = plsc.VectorSubcoreMesh(
        core_axis_name="c", subcore_axis_name="s",
        num_cores=_NC, num_subcores=_NS)

    @functools.partial(
        pl.kernel,
        mesh=mesh,
        compiler_params=pltpu.CompilerParams(use_tc_tiling_on_sc=False),
        out_type=jax.ShapeDtypeStruct((B, P, H), embeddings.dtype),
        scratch_types=[
            pltpu.VMEM((2, _R, H), jnp.float32),
            pltpu.VMEM((_NBUF, _R, H), jnp.float32),
            pltpu.SemaphoreType.DMA,
            pltpu.SemaphoreType.DMA,
            pltpu.SemaphoreType.DMA,
        ],
    )
    def add_pos(emb_hbm, pos_hbm, out_hbm, pos_buf, ebuf, sem_pos, sem_in, sem_out):
        wid = lax.axis_index("s") * _NC + lax.axis_index("c")
        base0 = wid * ppw

        def start_in(j):
            ch = j >> 2
            b = j & 3
            pltpu.async_copy(
                emb_hbm.at[b, pl.ds(base0 + ch * _R, _R)],
                ebuf.at[j & (_NBUF - 1)], sem_in)

        def wait_in():
            pltpu.make_async_copy(
                emb_hbm.at[0, pl.ds(0, _R)], ebuf.at[0], sem_in).wait()

        def start_out(j):
            ch = j >> 2
            b = j & 3
            pltpu.async_copy(
                ebuf.at[j & (_NBUF - 1)],
                out_hbm.at[b, pl.ds(base0 + ch * _R, _R)], sem_out)

        def wait_out():
            pltpu.make_async_copy(
                ebuf.at[0], out_hbm.at[0, pl.ds(0, _R)], sem_out).wait()

        def start_pos(ch):
            pltpu.async_copy(
                pos_hbm.at[pl.ds(base0 + ch * _R, _R)], pos_buf.at[ch % 2], sem_pos)

        def wait_pos():
            pltpu.make_async_copy(
                pos_hbm.at[pl.ds(0, _R)], pos_buf.at[0], sem_pos).wait()

        start_pos(0)
        for j in range(_LEAD):
            start_in(j)

        nunit = nchunk * B

        @pl.loop(0, nunit)
        def unit_loop(j):
            ch = j >> 2
            b = j & 3

            @pl.when(b == 0)
            def _():
                wait_pos()

                @pl.when(ch + 1 < nchunk)
                def _():
                    start_pos(ch + 1)

            wait_in()
            pos_ref = pos_buf.at[ch % 2]
            eref = ebuf.at[j & (_NBUF - 1)]

            @plsc.parallel_loop(0, nvec, unroll=8)
            def vbody(i):
                r = i & (_R - 1)
                coff = (i >> 4) * _L
                pv = pos_ref[r, pl.ds(coff, _L)]
                plsc.addupdate(eref.at[r, pl.ds(coff, _L)], pv)

            start_out(j)

            @pl.when(j >= _LEAD)
            def _():
                wait_out()

            @pl.when(j + _LEAD < nunit)
            def _():
                start_in(j + _LEAD)

        for _ in range(_LEAD):
            wait_out()

    return add_pos(embeddings, pos_table)


# speedup vs baseline: 3.2650x; 3.2650x over previous
import functools

import jax
import jax.numpy as jnp
from jax import lax
from jax.experimental import pallas as pl
from jax.experimental.pallas import tpu as pltpu
from jax.experimental.pallas import tpu_sc as plsc

_NC = 2
_NS = 16
_NW = _NC * _NS
_L = 16
_R = 16
_NBUF = 8
_LEAD = 6


def kernel(embeddings, pos_table):
    B, P, H = embeddings.shape
    ppw = P // _NW
    nchunk = ppw // _R
    nvec = (_R * H) // _L

    mesh = plsc.VectorSubcoreMesh(
        core_axis_name="c", subcore_axis_name="s",
        num_cores=_NC, num_subcores=_NS)

    @functools.partial(
        pl.kernel,
        mesh=mesh,
        out_type=jax.ShapeDtypeStruct((B, P, H), embeddings.dtype),
        scratch_types=[
            pltpu.VMEM((2, _R, H), jnp.float32),
            pltpu.VMEM((_NBUF, _R, H), jnp.float32),
            pltpu.SemaphoreType.DMA,
            pltpu.SemaphoreType.DMA,
            pltpu.SemaphoreType.DMA,
        ],
    )
    def add_pos(emb_hbm, pos_hbm, out_hbm, pos_buf, ebuf, sem_pos, sem_in, sem_out):
        wid = lax.axis_index("s") * _NC + lax.axis_index("c")
        base0 = wid * ppw

        def start_in(j):
            ch = j >> 2
            b = j & 3
            pltpu.async_copy(
                emb_hbm.at[b, pl.ds(base0 + ch * _R, _R)],
                ebuf.at[j & (_NBUF - 1)], sem_in)

        def wait_in():
            pltpu.make_async_copy(
                emb_hbm.at[0, pl.ds(0, _R)], ebuf.at[0], sem_in).wait()

        def start_out(j):
            ch = j >> 2
            b = j & 3
            pltpu.async_copy(
                ebuf.at[j & (_NBUF - 1)],
                out_hbm.at[b, pl.ds(base0 + ch * _R, _R)], sem_out)

        def wait_out():
            pltpu.make_async_copy(
                ebuf.at[0], out_hbm.at[0, pl.ds(0, _R)], sem_out).wait()

        def start_pos(ch):
            pltpu.async_copy(
                pos_hbm.at[pl.ds(base0 + ch * _R, _R)], pos_buf.at[ch % 2], sem_pos)

        def wait_pos():
            pltpu.make_async_copy(
                pos_hbm.at[pl.ds(0, _R)], pos_buf.at[0], sem_pos).wait()

        start_pos(0)
        for j in range(_LEAD):
            start_in(j)

        nunit = nchunk * B

        @pl.loop(0, nunit)
        def unit_loop(j):
            ch = j >> 2
            b = j & 3

            @pl.when(b == 0)
            def _():
                wait_pos()

                @pl.when(ch + 1 < nchunk)
                def _():
                    start_pos(ch + 1)

            wait_in()
            pos_ref = pos_buf.at[ch % 2]
            eref = ebuf.at[j & (_NBUF - 1)]

            @plsc.parallel_loop(0, nvec, unroll=16)
            def vbody(i):
                r = i & (_R - 1)
                coff = (i >> 4) * _L
                pv = pos_ref[r, pl.ds(coff, _L)]
                plsc.addupdate(eref.at[r, pl.ds(coff, _L)], pv)

            start_out(j)

            @pl.when(j >= _LEAD)
            def _():
                wait_out()

            @pl.when(j + _LEAD < nunit)
            def _():
                start_in(j + _LEAD)

        for _ in range(_LEAD):
            wait_out()

    return add_pos(embeddings, pos_table)


# speedup vs baseline: 3.3012x; 1.0111x over previous
import functools

import jax
import jax.numpy as jnp
from jax import lax
from jax.experimental import pallas as pl
from jax.experimental.pallas import tpu as pltpu
from jax.experimental.pallas import tpu_sc as plsc

_NC = 2
_NS = 16
_NW = _NC * _NS
_L = 16
_R = 16
_NBUF = 8
_LEAD = 4


def kernel(embeddings, pos_table):
    B, P, H = embeddings.shape
    ppw = P // _NW
    nchunk = ppw // _R
    nvec = (_R * H) // _L

    mesh = plsc.VectorSubcoreMesh(
        core_axis_name="c", subcore_axis_name="s",
        num_cores=_NC, num_subcores=_NS)

    @functools.partial(
        pl.kernel,
        mesh=mesh,
        out_type=jax.ShapeDtypeStruct((B, P, H), embeddings.dtype),
        scratch_types=[
            pltpu.VMEM((2, _R, H), jnp.float32),
            pltpu.VMEM((_NBUF, _R, H), jnp.float32),
            pltpu.SemaphoreType.DMA,
            pltpu.SemaphoreType.DMA,
            pltpu.SemaphoreType.DMA,
        ],
    )
    def add_pos(emb_hbm, pos_hbm, out_hbm, pos_buf, ebuf, sem_pos, sem_in, sem_out):
        wid = lax.axis_index("c") * _NS + lax.axis_index("s")
        base0 = wid * ppw

        def start_in(j):
            ch = j >> 2
            b = j & 3
            pltpu.async_copy(
                emb_hbm.at[b, pl.ds(base0 + ch * _R, _R)],
                ebuf.at[j & (_NBUF - 1)], sem_in)

        def wait_in():
            pltpu.make_async_copy(
                emb_hbm.at[0, pl.ds(0, _R)], ebuf.at[0], sem_in).wait()

        def start_out(j):
            ch = j >> 2
            b = j & 3
            pltpu.async_copy(
                ebuf.at[j & (_NBUF - 1)],
                out_hbm.at[b, pl.ds(base0 + ch * _R, _R)], sem_out)

        def wait_out():
            pltpu.make_async_copy(
                ebuf.at[0], out_hbm.at[0, pl.ds(0, _R)], sem_out).wait()

        def start_pos(ch):
            pltpu.async_copy(
                pos_hbm.at[pl.ds(base0 + ch * _R, _R)], pos_buf.at[ch % 2], sem_pos)

        def wait_pos():
            pltpu.make_async_copy(
                pos_hbm.at[pl.ds(0, _R)], pos_buf.at[0], sem_pos).wait()

        start_pos(0)
        for j in range(_LEAD):
            start_in(j)

        nunit = nchunk * B

        @pl.loop(0, nunit)
        def unit_loop(j):
            ch = j >> 2
            b = j & 3

            @pl.when(b == 0)
            def _():
                wait_pos()

                @pl.when(ch + 1 < nchunk)
                def _():
                    start_pos(ch + 1)

            wait_in()
            pos_ref = pos_buf.at[ch % 2]
            eref = ebuf.at[j & (_NBUF - 1)]

            @plsc.parallel_loop(0, nvec, unroll=8)
            def vbody(i):
                r = i & (_R - 1)
                coff = (i >> 4) * _L
                pv = pos_ref[r, pl.ds(coff, _L)]
                plsc.addupdate(eref.at[r, pl.ds(coff, _L)], pv)

            start_out(j)

            @pl.when(j >= _LEAD)
            def _():
                wait_out()

            @pl.when(j + _LEAD < nunit)
            def _():
                start_in(j + _LEAD)

        for _ in range(_LEAD):
            wait_out()

    return add_pos(embeddings, pos_table)
